# baseline (speedup 1.0000x reference)
import functools

import jax
import jax.numpy as jnp
from jax import lax
from jax.experimental import pallas as pl
from jax.experimental.pallas import tpu as pltpu
from jax.experimental.pallas import tpu_sc as plsc

N = 50000
E = 1600000
H = 256
G = 64

NC = 2
NS = 16
NW = NC * NS
LW = 128
NP = 51200
SLICE = NP // NS
ROWS_TOTAL = E // LW
CR = 8
NCH_FULL = ROWS_TOTAL // CR
NCH_BASE = NCH_FULL // NW
NCH_EXTRA = NCH_FULL % NW
TAIL_ROWS = ROWS_TOTAL - NCH_FULL * CR
TAIL_RB = NCH_FULL * CR

BN = 2048
NB = NP // BN

_mesh = plsc.VectorSubcoreMesh(core_axis_name="c", subcore_axis_name="s",
                               num_cores=NC, num_subcores=NS)
_sc_params = pltpu.CompilerParams(use_tc_tiling_on_sc=False)


_DEG_SCRATCH = [
    pltpu.VMEM((CR, LW), jnp.int32),
    pltpu.VMEM((LW,), jnp.float32),
    pltpu.VMEM((SLICE,), jnp.float32),
    pltpu.VMEM_SHARED((NP,), jnp.float32),
    pltpu.SemaphoreType.DMA,
]


def _deg_body(er_hbm, out_hbm, idx_v, ones_v, zbuf_v, acc_sh, sem):
    cid = lax.axis_index("c")
    sid = lax.axis_index("s")
    wid = sid * NC + cid

    for i in range(LW // 16):
        ones_v[pl.ds(i * 16, 16)] = jnp.full((16,), 1.0, jnp.float32)
    for i in range(SLICE // 16):
        zbuf_v[pl.ds(i * 16, 16)] = jnp.zeros((16,), jnp.float32)

    tslice = pl.ds(pl.multiple_of(sid * SLICE, 8), SLICE)
    pltpu.sync_copy(zbuf_v, acc_sh.at[tslice])
    plsc.subcore_barrier()

    nch = jnp.where(wid < NCH_EXTRA, NCH_BASE + 1, NCH_BASE)

    def chunk(g, carry):
        rb = pl.multiple_of((wid + NW * g) * CR, 8)
        pltpu.sync_copy(er_hbm.at[1, pl.ds(rb, CR), :], idx_v)
        descs = [
            pltpu.async_copy(ones_v, acc_sh.at[idx_v.at[j]], sem, add=True)
            for j in range(CR)
        ]
        for d in descs:
            d.wait()
        return carry

    lax.fori_loop(0, nch, chunk, 0)

    @pl.when(wid == NW - 1)
    def _():
        pltpu.sync_copy(er_hbm.at[1, pl.ds(TAIL_RB, TAIL_ROWS), :],
                        idx_v.at[pl.ds(0, TAIL_ROWS), :])
        descs = [
            pltpu.async_copy(ones_v, acc_sh.at[idx_v.at[j]], sem, add=True)
            for j in range(TAIL_ROWS)
        ]
        for d in descs:
            d.wait()

    plsc.subcore_barrier()
    oslice = pl.ds(pl.multiple_of(cid * NP + sid * SLICE, 8), SLICE)
    pltpu.sync_copy(acc_sh.at[tslice], out_hbm.at[oslice])


_deg_kernel = pl.kernel(
    _deg_body,
    out_type=jax.ShapeDtypeStruct((NC * NP,), jnp.float32),
    mesh=_mesh,
    scratch_types=_DEG_SCRATCH,
    compiler_params=_sc_params,
)


_AGG_SCRATCH = [
    pltpu.VMEM((2, CR, LW), jnp.int32),
    pltpu.VMEM((2, CR, LW), jnp.int32),
    pltpu.VMEM((2, CR, LW), jnp.float32),
    pltpu.VMEM((2, CR, LW), jnp.float32),
    pltpu.VMEM((SLICE,), jnp.float32),
    pltpu.VMEM((SLICE,), jnp.float32),
    pltpu.VMEM((SLICE,), jnp.float32),
    pltpu.VMEM((SLICE,), jnp.float32),
    pltpu.VMEM_SHARED((NP,), jnp.float32),
    pltpu.VMEM_SHARED((NP,), jnp.float32),
    pltpu.VMEM_SHARED((NP,), jnp.float32),
    pltpu.VMEM_SHARED((NP,), jnp.float32),
    pltpu.SemaphoreType.DMA,
    pltpu.SemaphoreType.DMA,
]


def _agg_body(er_hbm, degp_hbm, x0_hbm, x1_hbm,
              dis_hbm, out0_hbm, out1_hbm,
              idxs_v, idxd_v, rows0_v, rows1_v, t0_v, t1_v, t2_v, t3_v,
              ctab0_sh, ctab1_sh, acc0_sh, acc1_sh, sem_g, sem_s):
    cid = lax.axis_index("c")
    sid = lax.axis_index("s")
    wid = sid * NC + cid

    nb = pl.multiple_of(sid * SLICE, 8)
    tslice = pl.ds(nb, SLICE)
    pltpu.sync_copy(degp_hbm.at[tslice], t0_v)
    pltpu.sync_copy(degp_hbm.at[pl.ds(pl.multiple_of(NP + nb, 8), SLICE)],
                    t1_v)
    pltpu.sync_copy(x0_hbm.at[tslice], t2_v)
    pltpu.sync_copy(x1_hbm.at[tslice], t3_v)

    def disc(i, carry):
        s = pl.ds(pl.multiple_of(i * 16, 16), 16)
        deg = t0_v[s] + t1_v[s] + 1.0
        bits = lax.bitcast_convert_type(deg, jnp.int32)
        bits = jnp.int32(0x5F3759DF) - lax.shift_right_logical(bits, 1)
        y = lax.bitcast_convert_type(bits, jnp.float32)
        y = y * (1.5 - 0.5 * deg * y * y)
        y = y * (1.5 - 0.5 * deg * y * y)
        y = y * (1.5 - 0.5 * deg * y * y)
        t0_v[s] = y
        t2_v[s] = t2_v[s] * y
        t3_v[s] = t3_v[s] * y
        return carry

    lax.fori_loop(0, SLICE // 16, disc, 0)

    @pl.when(cid == 0)
    def _():
        pltpu.sync_copy(t0_v, dis_hbm.at[tslice])

    pltpu.sync_copy(t2_v, ctab0_sh.at[tslice])
    pltpu.sync_copy(t3_v, ctab1_sh.at[tslice])

    @pl.when(cid == 0)
    def _():
        pltpu.sync_copy(t2_v, acc0_sh.at[tslice])
        pltpu.sync_copy(t3_v, acc1_sh.at[tslice])

    @pl.when(cid == 1)
    def _():
        def zf(i, carry):
            t1_v[pl.ds(pl.multiple_of(i * 16, 16), 16)] = jnp.zeros(
                (16,), jnp.float32)
            return carry

        lax.fori_loop(0, SLICE // 16, zf, 0)
        pltpu.sync_copy(t1_v, acc0_sh.at[tslice])
        pltpu.sync_copy(t1_v, acc1_sh.at[tslice])

    plsc.subcore_barrier()

    def drain_scatters(p):
        for j in range(CR):
            pltpu.make_async_copy(
                rows0_v.at[p, j], acc0_sh.at[idxd_v.at[p, j]], sem_s).wait()
            pltpu.make_async_copy(
                rows1_v.at[p, j], acc1_sh.at[idxd_v.at[p, j]], sem_s).wait()

    def step(p, q, first):
        if not first:
            drain_scatters(p)
        rb = pl.multiple_of(q * CR, 8)
        pltpu.sync_copy(er_hbm.at[0, pl.ds(rb, CR), :], idxs_v.at[p])
        pltpu.sync_copy(er_hbm.at[1, pl.ds(rb, CR), :], idxd_v.at[p])
        gds = [
            pltpu.async_copy(ctab0_sh.at[idxs_v.at[p, j]], rows0_v.at[p, j],
                             sem_g)
            for j in range(CR)
        ] + [
            pltpu.async_copy(ctab1_sh.at[idxs_v.at[p, j]], rows1_v.at[p, j],
                             sem_g)
            for j in range(CR)
        ]
        for d in gds:
            d.wait()
        for j in range(CR):
            pltpu.async_copy(rows0_v.at[p, j], acc0_sh.at[idxd_v.at[p, j]],
                             sem_s, add=True)
            pltpu.async_copy(rows1_v.at[p, j], acc1_sh.at[idxd_v.at[p, j]],
                             sem_s, add=True)

    step(0, wid, True)
    step(1, wid + NW, True)

    def pairbody(g2, carry):
        step(0, wid + NW * (2 * g2), False)
        step(1, wid + NW * (2 * g2 + 1), False)
        return carry

    lax.fori_loop(1, NCH_BASE // 2, pairbody, 0)

    @pl.when(wid < NCH_EXTRA)
    def _():
        step(0, wid + NW * NCH_BASE, False)

    drain_scatters(0)
    drain_scatters(1)

    @pl.when(wid == NW - 1)
    def _():
        pltpu.sync_copy(er_hbm.at[0, pl.ds(TAIL_RB, TAIL_ROWS), :],
                        idxs_v.at[0, pl.ds(0, TAIL_ROWS), :])
        pltpu.sync_copy(er_hbm.at[1, pl.ds(TAIL_RB, TAIL_ROWS), :],
                        idxd_v.at[0, pl.ds(0, TAIL_ROWS), :])
        tg = [
            pltpu.async_copy(ctab0_sh.at[idxs_v.at[0, j]], rows0_v.at[0, j],
                             sem_g)
            for j in range(TAIL_ROWS)
        ] + [
            pltpu.async_copy(ctab1_sh.at[idxs_v.at[0, j]], rows1_v.at[0, j],
                             sem_g)
            for j in range(TAIL_ROWS)
        ]
        for d in tg:
            d.wait()
        ts = [
            pltpu.async_copy(rows0_v.at[0, j], acc0_sh.at[idxd_v.at[0, j]],
                             sem_s, add=True)
            for j in range(TAIL_ROWS)
        ] + [
            pltpu.async_copy(rows1_v.at[0, j], acc1_sh.at[idxd_v.at[0, j]],
                             sem_s, add=True)
            for j in range(TAIL_ROWS)
        ]
        for d in ts:
            d.wait()

    plsc.subcore_barrier()
    oslice = pl.ds(pl.multiple_of(cid * NP + sid * SLICE, 8), SLICE)
    pltpu.sync_copy(acc0_sh.at[tslice], out0_hbm.at[oslice])
    pltpu.sync_copy(acc1_sh.at[tslice], out1_hbm.at[oslice])


_agg_kernel = pl.kernel(
    _agg_body,
    out_type=[
        jax.ShapeDtypeStruct((NP,), jnp.float32),
        jax.ShapeDtypeStruct((NC * NP,), jnp.float32),
        jax.ShapeDtypeStruct((NC * NP,), jnp.float32),
    ],
    mesh=_mesh,
    scratch_types=_AGG_SCRATCH,
    compiler_params=_sc_params,
)


def _pool_body(o0_ref, o1_ref, dis_ref, batch_ref, w1t_ref, b1_ref,
               wlt_ref, bl_ref, out_ref, pool_acc, cnt_acc):
    i = pl.program_id(0)

    @pl.when(i == 0)
    def _():
        pool_acc[...] = jnp.zeros_like(pool_acc)
        cnt_acc[...] = jnp.zeros_like(cnt_acc)

    dis = dis_ref[...]
    dt0 = (o0_ref[0:1, :] + o0_ref[1:2, :]) * dis
    dt1 = (o1_ref[0:1, :] + o1_ref[1:2, :]) * dis
    w1t = w1t_ref[...]
    h2 = jnp.maximum(
        w1t[:, 0:1] * dt0 + w1t[:, 1:2] * dt1 + b1_ref[...],
        0.0,
    )
    gids = lax.broadcasted_iota(jnp.int32, (G, BN), 0)
    oh = (gids == batch_ref[...]).astype(jnp.float32)
    pool_acc[...] += lax.dot_general(
        h2, oh, (((1,), (1,)), ((), ())),
        precision=lax.Precision.HIGHEST,
        preferred_element_type=jnp.float32)
    cnt_acc[...] += lax.dot_general(
        jnp.ones((1, BN), jnp.float32), oh, (((1,), (1,)), ((), ())),
        precision=lax.Precision.HIGHEST,
        preferred_element_type=jnp.float32)

    @pl.when(i == NB - 1)
    def _():
        pooled = pool_acc[...] / jnp.maximum(cnt_acc[...], 1.0)
        num = jnp.dot(wlt_ref[...], pooled,
                      precision=lax.Precision.HIGHEST,
                      preferred_element_type=jnp.float32)
        out_ref[...] = num + bl_ref[...]


_pool_call = pl.pallas_call(
    _pool_body,
    grid=(NB,),
    in_specs=[
        pl.BlockSpec((2, BN), lambda i: (0, i)),
        pl.BlockSpec((2, BN), lambda i: (0, i)),
        pl.BlockSpec((1, BN), lambda i: (0, i)),
        pl.BlockSpec((1, BN), lambda i: (0, i)),
        pl.BlockSpec((H, 2), lambda i: (0, 0)),
        pl.BlockSpec((H, 1), lambda i: (0, 0)),
        pl.BlockSpec((1, H), lambda i: (0, 0)),
        pl.BlockSpec((1, 1), lambda i: (0, 0)),
    ],
    out_specs=pl.BlockSpec((1, G), lambda i: (0, 0)),
    out_shape=jax.ShapeDtypeStruct((1, G), jnp.float32),
    scratch_shapes=[
        pltpu.VMEM((H, G), jnp.float32),
        pltpu.VMEM((1, G), jnp.float32),
    ],
)


def kernel(x, edge_index, batch, W1, b1, Wl, bl):
    er = edge_index.reshape(2, ROWS_TOTAL, LW)

    degp = _deg_kernel(er)

    x0 = jnp.zeros((NP,), jnp.float32).at[:N].set(x[:, 0])
    x1 = jnp.zeros((NP,), jnp.float32).at[:N].set(x[:, 1])
    dis, o0, o1 = _agg_kernel(er, degp, x0, x1)

    batch_pad = jnp.full((1, NP), G, jnp.int32).at[0, :N].set(batch)
    out_row = _pool_call(
        o0.reshape(NC, NP),
        o1.reshape(NC, NP),
        dis.reshape(1, NP),
        batch_pad,
        W1.T,
        b1[:, None],
        Wl.T,
        bl[None, :],
    )
    return out_row.reshape(G, 1)


# speedup vs baseline: 1.0277x; 1.0277x over previous
import functools

import jax
import jax.numpy as jnp
from jax import lax
from jax.experimental import pallas as pl
from jax.experimental.pallas import tpu as pltpu
from jax.experimental.pallas import tpu_sc as plsc

N = 50000
E = 1600000
H = 256
G = 64

NC = 2
NS = 16
NW = NC * NS
LW = 128
NP = 51200
SLICE = NP // NS
ROWS_TOTAL = E // LW
CR = 8
NCH_FULL = ROWS_TOTAL // CR
NCH_BASE = NCH_FULL // NW
NCH_EXTRA = NCH_FULL % NW
TAIL_ROWS = ROWS_TOTAL - NCH_FULL * CR
TAIL_RB = NCH_FULL * CR

BN = 2048
NB = NP // BN

_mesh = plsc.VectorSubcoreMesh(core_axis_name="c", subcore_axis_name="s",
                               num_cores=NC, num_subcores=NS)
_sc_params = pltpu.CompilerParams(use_tc_tiling_on_sc=False)


_DEG_SCRATCH = [
    pltpu.VMEM((CR, LW), jnp.int32),
    pltpu.VMEM((LW,), jnp.float32),
    pltpu.VMEM((SLICE,), jnp.float32),
    pltpu.VMEM_SHARED((NP,), jnp.float32),
    pltpu.SemaphoreType.DMA,
]


def _deg_body(er_hbm, out_hbm, idx_v, ones_v, zbuf_v, acc_sh, sem):
    cid = lax.axis_index("c")
    sid = lax.axis_index("s")
    wid = sid * NC + cid

    for i in range(LW // 16):
        ones_v[pl.ds(i * 16, 16)] = jnp.full((16,), 1.0, jnp.float32)
    for i in range(SLICE // 16):
        zbuf_v[pl.ds(i * 16, 16)] = jnp.zeros((16,), jnp.float32)

    tslice = pl.ds(pl.multiple_of(sid * SLICE, 8), SLICE)
    pltpu.sync_copy(zbuf_v, acc_sh.at[tslice])
    plsc.subcore_barrier()

    nch = jnp.where(wid < NCH_EXTRA, NCH_BASE + 1, NCH_BASE)

    def chunk(g, carry):
        rb = pl.multiple_of((wid + NW * g) * CR, 8)
        pltpu.sync_copy(er_hbm.at[1, pl.ds(rb, CR), :], idx_v)
        descs = [
            pltpu.async_copy(ones_v, acc_sh.at[idx_v.at[j]], sem, add=True)
            for j in range(CR)
        ]
        for d in descs:
            d.wait()
        return carry

    lax.fori_loop(0, nch, chunk, 0)

    @pl.when(wid == NW - 1)
    def _():
        pltpu.sync_copy(er_hbm.at[1, pl.ds(TAIL_RB, TAIL_ROWS), :],
                        idx_v.at[pl.ds(0, TAIL_ROWS), :])
        descs = [
            pltpu.async_copy(ones_v, acc_sh.at[idx_v.at[j]], sem, add=True)
            for j in range(TAIL_ROWS)
        ]
        for d in descs:
            d.wait()

    plsc.subcore_barrier()
    oslice = pl.ds(pl.multiple_of(cid * NP + sid * SLICE, 8), SLICE)
    pltpu.sync_copy(acc_sh.at[tslice], out_hbm.at[oslice])


_deg_kernel = pl.kernel(
    _deg_body,
    out_type=jax.ShapeDtypeStruct((NC * NP,), jnp.float32),
    mesh=_mesh,
    scratch_types=_DEG_SCRATCH,
    compiler_params=_sc_params,
)


_AGG_SCRATCH = [
    pltpu.VMEM((2, CR, LW), jnp.int32),
    pltpu.VMEM((2, CR, LW), jnp.int32),
    pltpu.VMEM((2, CR, LW), jnp.float32),
    pltpu.VMEM((2, CR, LW), jnp.float32),
    pltpu.VMEM((SLICE,), jnp.float32),
    pltpu.VMEM((SLICE,), jnp.float32),
    pltpu.VMEM((SLICE,), jnp.float32),
    pltpu.VMEM((SLICE,), jnp.float32),
    pltpu.VMEM_SHARED((NP,), jnp.float32),
    pltpu.VMEM_SHARED((NP,), jnp.float32),
    pltpu.VMEM_SHARED((NP,), jnp.float32),
    pltpu.VMEM_SHARED((NP,), jnp.float32),
    pltpu.SemaphoreType.DMA,
    pltpu.SemaphoreType.DMA,
]


def _agg_body(er_hbm, degp_hbm, x0_hbm, x1_hbm,
              dis_hbm, out0_hbm, out1_hbm,
              idxs_v, idxd_v, rows0_v, rows1_v, t0_v, t1_v, t2_v, t3_v,
              ctab0_sh, ctab1_sh, acc0_sh, acc1_sh, sem_g, sem_s):
    cid = lax.axis_index("c")
    sid = lax.axis_index("s")
    wid = sid * NC + cid

    nb = pl.multiple_of(sid * SLICE, 8)
    tslice = pl.ds(nb, SLICE)
    pltpu.sync_copy(degp_hbm.at[tslice], t0_v)
    pltpu.sync_copy(degp_hbm.at[pl.ds(pl.multiple_of(NP + nb, 8), SLICE)],
                    t1_v)
    pltpu.sync_copy(x0_hbm.at[tslice], t2_v)
    pltpu.sync_copy(x1_hbm.at[tslice], t3_v)

    def disc(i, carry):
        s = pl.ds(pl.multiple_of(i * 16, 16), 16)
        deg = t0_v[s] + t1_v[s] + 1.0
        bits = lax.bitcast_convert_type(deg, jnp.int32)
        bits = jnp.int32(0x5F3759DF) - lax.shift_right_logical(bits, 1)
        y = lax.bitcast_convert_type(bits, jnp.float32)
        y = y * (1.5 - 0.5 * deg * y * y)
        y = y * (1.5 - 0.5 * deg * y * y)
        y = y * (1.5 - 0.5 * deg * y * y)
        t0_v[s] = y
        t2_v[s] = t2_v[s] * y
        t3_v[s] = t3_v[s] * y
        return carry

    lax.fori_loop(0, SLICE // 16, disc, 0)

    @pl.when(cid == 0)
    def _():
        pltpu.sync_copy(t0_v, dis_hbm.at[tslice])

    pltpu.sync_copy(t2_v, ctab0_sh.at[tslice])
    pltpu.sync_copy(t3_v, ctab1_sh.at[tslice])

    @pl.when(cid == 0)
    def _():
        pltpu.sync_copy(t2_v, acc0_sh.at[tslice])
        pltpu.sync_copy(t3_v, acc1_sh.at[tslice])

    @pl.when(cid == 1)
    def _():
        def zf(i, carry):
            t1_v[pl.ds(pl.multiple_of(i * 16, 16), 16)] = jnp.zeros(
                (16,), jnp.float32)
            return carry

        lax.fori_loop(0, SLICE // 16, zf, 0)
        pltpu.sync_copy(t1_v, acc0_sh.at[tslice])
        pltpu.sync_copy(t1_v, acc1_sh.at[tslice])

    plsc.subcore_barrier()

    def drain_scatters(p):
        for j in range(CR):
            pltpu.make_async_copy(
                rows0_v.at[p, j], acc0_sh.at[idxd_v.at[p, j]], sem_s).wait()
            pltpu.make_async_copy(
                rows1_v.at[p, j], acc1_sh.at[idxd_v.at[p, j]], sem_s).wait()

    def step(p, q, first):
        if not first:
            drain_scatters(p)
        rb = pl.multiple_of(q * CR, 8)
        pltpu.sync_copy(er_hbm.at[0, pl.ds(rb, CR), :], idxs_v.at[p])
        pltpu.sync_copy(er_hbm.at[1, pl.ds(rb, CR), :], idxd_v.at[p])
        gds = [
            pltpu.async_copy(ctab0_sh.at[idxs_v.at[p, j]], rows0_v.at[p, j],
                             sem_g)
            for j in range(CR)
        ] + [
            pltpu.async_copy(ctab1_sh.at[idxs_v.at[p, j]], rows1_v.at[p, j],
                             sem_g)
            for j in range(CR)
        ]
        for d in gds:
            d.wait()
        for j in range(CR):
            pltpu.async_copy(rows0_v.at[p, j], acc0_sh.at[idxd_v.at[p, j]],
                             sem_s, add=True)
            pltpu.async_copy(rows1_v.at[p, j], acc1_sh.at[idxd_v.at[p, j]],
                             sem_s, add=True)

    step(0, wid, True)
    step(1, wid + NW, True)

    def pairbody(g2, carry):
        step(0, wid + NW * (2 * g2), False)
        step(1, wid + NW * (2 * g2 + 1), False)
        return carry

    lax.fori_loop(1, NCH_BASE // 2, pairbody, 0)

    @pl.when(wid < NCH_EXTRA)
    def _():
        step(0, wid + NW * NCH_BASE, False)

    drain_scatters(0)
    drain_scatters(1)

    @pl.when(wid == NW - 1)
    def _():
        pltpu.sync_copy(er_hbm.at[0, pl.ds(TAIL_RB, TAIL_ROWS), :],
                        idxs_v.at[0, pl.ds(0, TAIL_ROWS), :])
        pltpu.sync_copy(er_hbm.at[1, pl.ds(TAIL_RB, TAIL_ROWS), :],
                        idxd_v.at[0, pl.ds(0, TAIL_ROWS), :])
        tg = [
            pltpu.async_copy(ctab0_sh.at[idxs_v.at[0, j]], rows0_v.at[0, j],
                             sem_g)
            for j in range(TAIL_ROWS)
        ] + [
            pltpu.async_copy(ctab1_sh.at[idxs_v.at[0, j]], rows1_v.at[0, j],
                             sem_g)
            for j in range(TAIL_ROWS)
        ]
        for d in tg:
            d.wait()
        ts = [
            pltpu.async_copy(rows0_v.at[0, j], acc0_sh.at[idxd_v.at[0, j]],
                             sem_s, add=True)
            for j in range(TAIL_ROWS)
        ] + [
            pltpu.async_copy(rows1_v.at[0, j], acc1_sh.at[idxd_v.at[0, j]],
                             sem_s, add=True)
            for j in range(TAIL_ROWS)
        ]
        for d in ts:
            d.wait()

    plsc.subcore_barrier()
    oslice = pl.ds(pl.multiple_of(cid * NP + sid * SLICE, 8), SLICE)
    pltpu.sync_copy(acc0_sh.at[tslice], out0_hbm.at[oslice])
    pltpu.sync_copy(acc1_sh.at[tslice], out1_hbm.at[oslice])


_agg_kernel = pl.kernel(
    _agg_body,
    out_type=[
        jax.ShapeDtypeStruct((NP,), jnp.float32),
        jax.ShapeDtypeStruct((NC * NP,), jnp.float32),
        jax.ShapeDtypeStruct((NC * NP,), jnp.float32),
    ],
    mesh=_mesh,
    scratch_types=_AGG_SCRATCH,
    compiler_params=_sc_params,
)


def _pool_body(o0_ref, o1_ref, dis_ref, batch_ref, w1t_ref, b1_ref,
               wlt_ref, bl_ref, out_ref, pool_acc, cnt_acc):
    i = pl.program_id(0)

    @pl.when(i == 0)
    def _():
        pool_acc[...] = jnp.zeros_like(pool_acc)
        cnt_acc[...] = jnp.zeros_like(cnt_acc)

    dis = dis_ref[...]
    dt0 = (o0_ref[0:1, :] + o0_ref[1:2, :]) * dis
    dt1 = (o1_ref[0:1, :] + o1_ref[1:2, :]) * dis
    w1t = w1t_ref[...]
    h2 = jnp.maximum(
        w1t[:, 0:1] * dt0 + w1t[:, 1:2] * dt1 + b1_ref[...],
        0.0,
    )
    gids = lax.broadcasted_iota(jnp.int32, (G, BN), 0)
    oh = (gids == batch_ref[...]).astype(jnp.float32)
    pool_acc[...] += lax.dot_general(
        h2, oh, (((1,), (1,)), ((), ())),
        precision=lax.Precision.HIGHEST,
        preferred_element_type=jnp.float32)
    cnt_acc[...] += lax.dot_general(
        jnp.ones((1, BN), jnp.float32), oh, (((1,), (1,)), ((), ())),
        preferred_element_type=jnp.float32)

    @pl.when(i == NB - 1)
    def _():
        pooled = pool_acc[...] / jnp.maximum(cnt_acc[...], 1.0)
        num = jnp.dot(wlt_ref[...], pooled,
                      precision=lax.Precision.HIGHEST,
                      preferred_element_type=jnp.float32)
        out_ref[...] = num + bl_ref[...]


_pool_call = pl.pallas_call(
    _pool_body,
    grid=(NB,),
    in_specs=[
        pl.BlockSpec((2, BN), lambda i: (0, i)),
        pl.BlockSpec((2, BN), lambda i: (0, i)),
        pl.BlockSpec((1, BN), lambda i: (0, i)),
        pl.BlockSpec((1, BN), lambda i: (0, i)),
        pl.BlockSpec((H, 2), lambda i: (0, 0)),
        pl.BlockSpec((H, 1), lambda i: (0, 0)),
        pl.BlockSpec((1, H), lambda i: (0, 0)),
        pl.BlockSpec((1, 1), lambda i: (0, 0)),
    ],
    out_specs=pl.BlockSpec((1, G), lambda i: (0, 0)),
    out_shape=jax.ShapeDtypeStruct((1, G), jnp.float32),
    scratch_shapes=[
        pltpu.VMEM((H, G), jnp.float32),
        pltpu.VMEM((1, G), jnp.float32),
    ],
)


def kernel(x, edge_index, batch, W1, b1, Wl, bl):
    er = edge_index.reshape(2, ROWS_TOTAL, LW)

    degp = _deg_kernel(er)

    x0 = jnp.zeros((NP,), jnp.float32).at[:N].set(x[:, 0])
    x1 = jnp.zeros((NP,), jnp.float32).at[:N].set(x[:, 1])
    dis, o0, o1 = _agg_kernel(er, degp, x0, x1)

    batch_pad = jnp.full((1, NP), G, jnp.int32).at[0, :N].set(batch)
    out_row = _pool_call(
        o0.reshape(NC, NP),
        o1.reshape(NC, NP),
        dis.reshape(1, NP),
        batch_pad,
        W1.T,
        b1[:, None],
        Wl.T,
        bl[None, :],
    )
    return out_row.reshape(G, 1)
